# Initial kernel scaffold; baseline (speedup 1.0000x reference)
import jax
import jax.numpy as jnp
from jax.experimental import pallas as pl


def kernel(ids, numeric, table, W1, b1, W2, b2):
    raise NotImplementedError("write your pallas kernel here")


# speedup vs baseline: 1.3034x; 1.3034x over previous
import jax
import jax.numpy as jnp
from jax.experimental import pallas as pl
from jax.experimental.pallas import tpu as pltpu
from jax.experimental.pallas import tpu_sc as plsc

_EMBED = 128
_HIDDEN = 256
_GATHER_WINDOW = 128
_TOKEN_BLOCK = 1024


def _sc_gather(table, flat_ids):
    n = flat_ids.shape[0]
    idx = flat_ids.reshape(1, n)
    mesh = plsc.VectorSubcoreMesh(core_axis_name="core",
                                  subcore_axis_name="subcore")

    @pl.kernel(out_type=jax.ShapeDtypeStruct((n, _EMBED), table.dtype),
               mesh=mesh)
    def gather_kernel(tab_hbm, i_hbm, o_hbm):
        def body(i_vmem, o_vmem):
            pltpu.sync_copy(tab_hbm.at[i_vmem.at[0]], o_vmem)

        pltpu.emit_pipeline(
            body,
            grid=(n // _GATHER_WINDOW,),
            in_specs=[pl.BlockSpec((1, _GATHER_WINDOW), lambda i: (0, i))],
            out_specs=[pl.BlockSpec((_GATHER_WINDOW, _EMBED),
                                    lambda i: (i, 0))],
            core_axis_name=("core", "subcore"),
            dimension_semantics=(pltpu.PARALLEL,),
        )(i_hbm, o_hbm)

    return gather_kernel(table, idx)


def _mlp_block_kernel(emb_ref, num_ref, w1e_ref, w1n_ref, b1_ref, w2_ref,
                      b2_ref, out_ref):
    h = jnp.dot(emb_ref[...], w1e_ref[...],
                preferred_element_type=jnp.float32)
    h = h + jnp.dot(num_ref[...], w1n_ref[...],
                    preferred_element_type=jnp.float32)
    h = jnp.maximum(h + b1_ref[...], 0.0)
    o = jnp.dot(h, w2_ref[...], preferred_element_type=jnp.float32)
    out_ref[...] = jnp.maximum(o + b2_ref[...], 0.0)


def _tc_mlp(emb, num, w1e, w1n, b1, w2, b2):
    n = emb.shape[0]
    t = _TOKEN_BLOCK
    grid = (n // t,)
    return pl.pallas_call(
        _mlp_block_kernel,
        grid=grid,
        in_specs=[
            pl.BlockSpec((t, _EMBED), lambda i: (i, 0)),
            pl.BlockSpec((t, 4), lambda i: (i, 0)),
            pl.BlockSpec((_EMBED, _HIDDEN), lambda i: (0, 0)),
            pl.BlockSpec((4, _HIDDEN), lambda i: (0, 0)),
            pl.BlockSpec((1, _HIDDEN), lambda i: (0, 0)),
            pl.BlockSpec((_HIDDEN, _HIDDEN), lambda i: (0, 0)),
            pl.BlockSpec((1, _HIDDEN), lambda i: (0, 0)),
        ],
        out_specs=pl.BlockSpec((t, _HIDDEN), lambda i: (i, 0)),
        out_shape=jax.ShapeDtypeStruct((n, _HIDDEN), jnp.float32),
        compiler_params=pltpu.CompilerParams(
            dimension_semantics=("arbitrary",)),
    )(emb, num, w1e, w1n, b1, w2, b2)


def kernel(ids, numeric, table, W1, b1, W2, b2):
    B, L = ids.shape
    n = B * L
    emb = _sc_gather(table, ids.reshape(-1))
    out = _tc_mlp(emb, numeric.reshape(n, 4),
                  W1[:_EMBED], W1[_EMBED:], b1.reshape(1, _HIDDEN),
                  W2, b2.reshape(1, _HIDDEN))
    return out.reshape(B, L, _HIDDEN)


# speedup vs baseline: 2.5711x; 1.9726x over previous
import jax
import jax.numpy as jnp
from jax.experimental import pallas as pl
from jax.experimental.pallas import tpu as pltpu
from jax.experimental.pallas import tpu_sc as plsc

_EMBED = 128
_HIDDEN = 256
_GATHER_WINDOW = 128
_TOKEN_BLOCK = 1024


def _sc_gather(table, flat_ids):
    n = flat_ids.shape[0]
    idx = flat_ids.reshape(1, n)
    mesh = plsc.VectorSubcoreMesh(core_axis_name="core",
                                  subcore_axis_name="subcore")

    @pl.kernel(out_type=jax.ShapeDtypeStruct((n, _EMBED), table.dtype),
               mesh=mesh)
    def gather_kernel(tab_hbm, i_hbm, o_hbm):
        def body(i_vmem, o_vmem):
            pltpu.sync_copy(tab_hbm.at[i_vmem.at[0]], o_vmem)

        pltpu.emit_pipeline(
            body,
            grid=(n // _GATHER_WINDOW,),
            in_specs=[pl.BlockSpec((1, _GATHER_WINDOW), lambda i: (0, i))],
            out_specs=[pl.BlockSpec((_GATHER_WINDOW, _EMBED),
                                    lambda i: (i, 0))],
            core_axis_name=("core", "subcore"),
            dimension_semantics=(pltpu.PARALLEL,),
        )(i_hbm, o_hbm)

    return gather_kernel(table, idx)


def _mlp_block_kernel(emb_ref, num_ref, w1e_ref, w1n_ref, b1_ref, w2_ref,
                      b2_ref, out_ref):
    h = jnp.dot(emb_ref[...].astype(jnp.bfloat16), w1e_ref[...],
                preferred_element_type=jnp.float32)
    h = h + jnp.dot(num_ref[...].astype(jnp.bfloat16), w1n_ref[...],
                    preferred_element_type=jnp.float32)
    h = jnp.maximum(h + b1_ref[...], 0.0)
    o = jnp.dot(h.astype(jnp.bfloat16), w2_ref[...],
                preferred_element_type=jnp.float32)
    out_ref[...] = jnp.maximum(o + b2_ref[...], 0.0)


def _tc_mlp(emb, num, w1e, w1n, b1, w2, b2):
    n = emb.shape[0]
    t = _TOKEN_BLOCK
    return pl.pallas_call(
        _mlp_block_kernel,
        grid=(n // t,),
        in_specs=[
            pl.BlockSpec((t, _EMBED), lambda i: (i, 0)),
            pl.BlockSpec((t, 4), lambda i: (i, 0)),
            pl.BlockSpec((_EMBED, _HIDDEN), lambda i: (0, 0)),
            pl.BlockSpec((4, _HIDDEN), lambda i: (0, 0)),
            pl.BlockSpec((1, _HIDDEN), lambda i: (0, 0)),
            pl.BlockSpec((_HIDDEN, _HIDDEN), lambda i: (0, 0)),
            pl.BlockSpec((1, _HIDDEN), lambda i: (0, 0)),
        ],
        out_specs=pl.BlockSpec((t, _HIDDEN), lambda i: (i, 0)),
        out_shape=jax.ShapeDtypeStruct((n, _HIDDEN), jnp.float32),
        compiler_params=pltpu.CompilerParams(
            dimension_semantics=("arbitrary",)),
    )(emb, num, w1e, w1n, b1, w2, b2)


def kernel(ids, numeric, table, W1, b1, W2, b2):
    B, L = ids.shape
    n = B * L
    emb = _sc_gather(table, ids.T.reshape(-1))
    num_sm = numeric.transpose(1, 0, 2).reshape(n, 4)
    bf = jnp.bfloat16
    out = _tc_mlp(emb, num_sm,
                  W1[:_EMBED].astype(bf), W1[_EMBED:].astype(bf),
                  b1.reshape(1, _HIDDEN), W2.astype(bf),
                  b2.reshape(1, _HIDDEN))
    return out.reshape(L, B, _HIDDEN).transpose(1, 0, 2)


# speedup vs baseline: 2.5760x; 1.0019x over previous
import jax
import jax.numpy as jnp
from jax.experimental import pallas as pl
from jax.experimental.pallas import tpu as pltpu
from jax.experimental.pallas import tpu_sc as plsc

_EMBED = 128
_HIDDEN = 256
_GATHER_WINDOW = 128
_TOKEN_BLOCK = 1024


def _sc_gather(table, flat_ids):
    n = flat_ids.shape[0]
    idx = flat_ids.reshape(1, n)
    mesh = plsc.VectorSubcoreMesh(core_axis_name="core",
                                  subcore_axis_name="subcore")

    @pl.kernel(out_type=jax.ShapeDtypeStruct((n, _EMBED), table.dtype),
               mesh=mesh)
    def gather_kernel(tab_hbm, i_hbm, o_hbm):
        def body(i_vmem, o_vmem):
            pltpu.sync_copy(tab_hbm.at[i_vmem.at[0]], o_vmem)

        pltpu.emit_pipeline(
            body,
            grid=(n // _GATHER_WINDOW,),
            in_specs=[pl.BlockSpec((1, _GATHER_WINDOW), lambda i: (0, i))],
            out_specs=[pl.BlockSpec((_GATHER_WINDOW, _EMBED),
                                    lambda i: (i, 0))],
            core_axis_name=("core", "subcore"),
            dimension_semantics=(pltpu.PARALLEL,),
        )(i_hbm, o_hbm)

    return gather_kernel(table, idx)


def _mlp_block_kernel(emb_ref, num_ref, w1e_ref, w1n_ref, b1_ref, w2_ref,
                      b2_ref, out_ref):
    h = jnp.dot(emb_ref[...].astype(jnp.bfloat16), w1e_ref[...],
                preferred_element_type=jnp.float32)
    h = h + jnp.dot(num_ref[...].astype(jnp.bfloat16), w1n_ref[...],
                    preferred_element_type=jnp.float32)
    h = jnp.maximum(h + b1_ref[...], 0.0)
    o = jnp.dot(h.astype(jnp.bfloat16), w2_ref[...],
                preferred_element_type=jnp.float32)
    out_ref[...] = jnp.maximum(o + b2_ref[...], 0.0)


def _tc_mlp(emb, num, w1e, w1n, b1, w2, b2):
    n = emb.shape[0]
    t = _TOKEN_BLOCK
    return pl.pallas_call(
        _mlp_block_kernel,
        grid=(n // t,),
        in_specs=[
            pl.BlockSpec((t, _EMBED), lambda i: (i, 0)),
            pl.BlockSpec((t, 4), lambda i: (i, 0)),
            pl.BlockSpec((_EMBED, _HIDDEN), lambda i: (0, 0)),
            pl.BlockSpec((4, _HIDDEN), lambda i: (0, 0)),
            pl.BlockSpec((1, _HIDDEN), lambda i: (0, 0)),
            pl.BlockSpec((_HIDDEN, _HIDDEN), lambda i: (0, 0)),
            pl.BlockSpec((1, _HIDDEN), lambda i: (0, 0)),
        ],
        out_specs=pl.BlockSpec((t, _HIDDEN), lambda i: (i, 0)),
        out_shape=jax.ShapeDtypeStruct((n, _HIDDEN), jnp.float32),
        compiler_params=pltpu.CompilerParams(
            dimension_semantics=("parallel",)),
    )(emb, num, w1e, w1n, b1, w2, b2)


def kernel(ids, numeric, table, W1, b1, W2, b2):
    B, L = ids.shape
    n = B * L
    emb = _sc_gather(table, ids.T.reshape(-1))
    num_sm = numeric.transpose(1, 0, 2).reshape(n, 4)
    bf = jnp.bfloat16
    out = _tc_mlp(emb, num_sm,
                  W1[:_EMBED].astype(bf), W1[_EMBED:].astype(bf),
                  b1.reshape(1, _HIDDEN), W2.astype(bf),
                  b2.reshape(1, _HIDDEN))
    return out.reshape(L, B, _HIDDEN).transpose(1, 0, 2)


# speedup vs baseline: 3.1029x; 1.2045x over previous
import jax
import jax.numpy as jnp
from jax.experimental import pallas as pl
from jax.experimental.pallas import tpu as pltpu
from jax.experimental.pallas import tpu_sc as plsc

_EMBED = 128
_HIDDEN = 256
_GATHER_WINDOW = 128
_TOKEN_BLOCK = 2048


def _sc_gather(table, flat_ids):
    n = flat_ids.shape[0]
    idx = flat_ids.reshape(1, n)
    mesh = plsc.VectorSubcoreMesh(core_axis_name="core",
                                  subcore_axis_name="subcore")

    @pl.kernel(out_type=jax.ShapeDtypeStruct((n, _EMBED), table.dtype),
               mesh=mesh)
    def gather_kernel(tab_hbm, i_hbm, o_hbm):
        def body(i_vmem, o_vmem):
            pltpu.sync_copy(tab_hbm.at[i_vmem.at[0]], o_vmem)

        pltpu.emit_pipeline(
            body,
            grid=(n // _GATHER_WINDOW,),
            in_specs=[pl.BlockSpec((1, _GATHER_WINDOW), lambda i: (0, i))],
            out_specs=[pl.BlockSpec((_GATHER_WINDOW, _EMBED),
                                    lambda i: (i, 0))],
            core_axis_name=("core", "subcore"),
            dimension_semantics=(pltpu.PARALLEL,),
        )(i_hbm, o_hbm)

    return gather_kernel(table, idx)


def _mlp_block_kernel(emb_ref, num_ref, w1e_ref, w1n_ref, b1_ref, w2_ref,
                      b2_ref, out_ref):
    h = jnp.dot(emb_ref[...].astype(jnp.bfloat16), w1e_ref[...],
                preferred_element_type=jnp.float32)
    h = h + jnp.dot(num_ref[...].astype(jnp.bfloat16), w1n_ref[...],
                    preferred_element_type=jnp.float32)
    h = jnp.maximum(h + b1_ref[...], 0.0)
    o = jnp.dot(h.astype(jnp.bfloat16), w2_ref[...],
                preferred_element_type=jnp.float32)
    out_ref[...] = jnp.maximum(o + b2_ref[...], 0.0)


def _tc_mlp(emb, num, w1e, w1n, b1, w2, b2):
    n = emb.shape[0]
    t = _TOKEN_BLOCK
    return pl.pallas_call(
        _mlp_block_kernel,
        grid=(n // t,),
        in_specs=[
            pl.BlockSpec((t, _EMBED), lambda i: (i, 0)),
            pl.BlockSpec((t, 4), lambda i: (i, 0)),
            pl.BlockSpec((_EMBED, _HIDDEN), lambda i: (0, 0)),
            pl.BlockSpec((4, _HIDDEN), lambda i: (0, 0)),
            pl.BlockSpec((1, _HIDDEN), lambda i: (0, 0)),
            pl.BlockSpec((_HIDDEN, _HIDDEN), lambda i: (0, 0)),
            pl.BlockSpec((1, _HIDDEN), lambda i: (0, 0)),
        ],
        out_specs=pl.BlockSpec((t, _HIDDEN), lambda i: (i, 0)),
        out_shape=jax.ShapeDtypeStruct((n, _HIDDEN), jnp.float32),
        compiler_params=pltpu.CompilerParams(
            dimension_semantics=("parallel",)),
    )(emb, num, w1e, w1n, b1, w2, b2)


def kernel(ids, numeric, table, W1, b1, W2, b2):
    B, L = ids.shape
    n = B * L
    emb = _sc_gather(table, ids.T.reshape(-1))
    num_sm = numeric.transpose(1, 0, 2).reshape(n, 4)
    bf = jnp.bfloat16
    out = _tc_mlp(emb, num_sm,
                  W1[:_EMBED].astype(bf), W1[_EMBED:].astype(bf),
                  b1.reshape(1, _HIDDEN), W2.astype(bf),
                  b2.reshape(1, _HIDDEN))
    return out.reshape(L, B, _HIDDEN).transpose(1, 0, 2)


# speedup vs baseline: 3.4754x; 1.1200x over previous
import jax
import jax.numpy as jnp
from jax.experimental import pallas as pl
from jax.experimental.pallas import tpu as pltpu
from jax.experimental.pallas import tpu_sc as plsc

_EMBED = 128
_HIDDEN = 256
_GATHER_WINDOW = 128
_TOKEN_BLOCK = 4096


def _sc_gather(table, flat_ids):
    n = flat_ids.shape[0]
    idx = flat_ids.reshape(1, n)
    mesh = plsc.VectorSubcoreMesh(core_axis_name="core",
                                  subcore_axis_name="subcore")

    @pl.kernel(out_type=jax.ShapeDtypeStruct((n, _EMBED), table.dtype),
               mesh=mesh)
    def gather_kernel(tab_hbm, i_hbm, o_hbm):
        def body(i_vmem, o_vmem):
            pltpu.sync_copy(tab_hbm.at[i_vmem.at[0]], o_vmem)

        pltpu.emit_pipeline(
            body,
            grid=(n // _GATHER_WINDOW,),
            in_specs=[pl.BlockSpec((1, _GATHER_WINDOW), lambda i: (0, i))],
            out_specs=[pl.BlockSpec((_GATHER_WINDOW, _EMBED),
                                    lambda i: (i, 0))],
            core_axis_name=("core", "subcore"),
            dimension_semantics=(pltpu.PARALLEL,),
        )(i_hbm, o_hbm)

    return gather_kernel(table, idx)


def _mlp_block_kernel(emb_ref, num_ref, w1e_ref, w1n_ref, b1_ref, w2_ref,
                      b2_ref, out_ref):
    h = jnp.dot(emb_ref[...].astype(jnp.bfloat16), w1e_ref[...],
                preferred_element_type=jnp.float32)
    h = h + jnp.dot(num_ref[...].astype(jnp.bfloat16), w1n_ref[...],
                    preferred_element_type=jnp.float32)
    h = jnp.maximum(h + b1_ref[...], 0.0)
    o = jnp.dot(h.astype(jnp.bfloat16), w2_ref[...],
                preferred_element_type=jnp.float32)
    out_ref[...] = jnp.maximum(o + b2_ref[...], 0.0)


def _tc_mlp(emb, num, w1e, w1n, b1, w2, b2):
    n = emb.shape[0]
    t = _TOKEN_BLOCK
    return pl.pallas_call(
        _mlp_block_kernel,
        grid=(n // t,),
        in_specs=[
            pl.BlockSpec((t, _EMBED), lambda i: (i, 0)),
            pl.BlockSpec((t, 4), lambda i: (i, 0)),
            pl.BlockSpec((_EMBED, _HIDDEN), lambda i: (0, 0)),
            pl.BlockSpec((4, _HIDDEN), lambda i: (0, 0)),
            pl.BlockSpec((1, _HIDDEN), lambda i: (0, 0)),
            pl.BlockSpec((_HIDDEN, _HIDDEN), lambda i: (0, 0)),
            pl.BlockSpec((1, _HIDDEN), lambda i: (0, 0)),
        ],
        out_specs=pl.BlockSpec((t, _HIDDEN), lambda i: (i, 0)),
        out_shape=jax.ShapeDtypeStruct((n, _HIDDEN), jnp.float32),
        compiler_params=pltpu.CompilerParams(
            dimension_semantics=("parallel",)),
    )(emb, num, w1e, w1n, b1, w2, b2)


def kernel(ids, numeric, table, W1, b1, W2, b2):
    B, L = ids.shape
    n = B * L
    emb = _sc_gather(table, ids.T.reshape(-1))
    num_sm = numeric.transpose(1, 0, 2).reshape(n, 4)
    bf = jnp.bfloat16
    out = _tc_mlp(emb, num_sm,
                  W1[:_EMBED].astype(bf), W1[_EMBED:].astype(bf),
                  b1.reshape(1, _HIDDEN), W2.astype(bf),
                  b2.reshape(1, _HIDDEN))
    return out.reshape(L, B, _HIDDEN).transpose(1, 0, 2)


# speedup vs baseline: 3.6517x; 1.0507x over previous
import jax
import jax.numpy as jnp
from jax.experimental import pallas as pl
from jax.experimental.pallas import tpu as pltpu
from jax.experimental.pallas import tpu_sc as plsc

_EMBED = 128
_HIDDEN = 256
_GATHER_WINDOW = 128
_TOKEN_BLOCK = 8192


def _sc_gather(table, flat_ids):
    n = flat_ids.shape[0]
    idx = flat_ids.reshape(1, n)
    mesh = plsc.VectorSubcoreMesh(core_axis_name="core",
                                  subcore_axis_name="subcore")

    @pl.kernel(out_type=jax.ShapeDtypeStruct((n, _EMBED), table.dtype),
               mesh=mesh)
    def gather_kernel(tab_hbm, i_hbm, o_hbm):
        def body(i_vmem, o_vmem):
            pltpu.sync_copy(tab_hbm.at[i_vmem.at[0]], o_vmem)

        pltpu.emit_pipeline(
            body,
            grid=(n // _GATHER_WINDOW,),
            in_specs=[pl.BlockSpec((1, _GATHER_WINDOW), lambda i: (0, i))],
            out_specs=[pl.BlockSpec((_GATHER_WINDOW, _EMBED),
                                    lambda i: (i, 0))],
            core_axis_name=("core", "subcore"),
            dimension_semantics=(pltpu.PARALLEL,),
        )(i_hbm, o_hbm)

    return gather_kernel(table, idx)


def _mlp_block_kernel(emb_ref, num_ref, w1e_ref, w1n_ref, b1_ref, w2_ref,
                      b2_ref, out_ref):
    h = jnp.dot(emb_ref[...].astype(jnp.bfloat16), w1e_ref[...],
                preferred_element_type=jnp.float32)
    h = h + jnp.dot(num_ref[...].astype(jnp.bfloat16), w1n_ref[...],
                    preferred_element_type=jnp.float32)
    h = jnp.maximum(h + b1_ref[...], 0.0)
    o = jnp.dot(h.astype(jnp.bfloat16), w2_ref[...],
                preferred_element_type=jnp.float32)
    out_ref[...] = jnp.maximum(o + b2_ref[...], 0.0)


def _tc_mlp(emb, num, w1e, w1n, b1, w2, b2):
    n = emb.shape[0]
    t = _TOKEN_BLOCK
    return pl.pallas_call(
        _mlp_block_kernel,
        grid=(n // t,),
        in_specs=[
            pl.BlockSpec((t, _EMBED), lambda i: (i, 0)),
            pl.BlockSpec((t, 4), lambda i: (i, 0)),
            pl.BlockSpec((_EMBED, _HIDDEN), lambda i: (0, 0)),
            pl.BlockSpec((4, _HIDDEN), lambda i: (0, 0)),
            pl.BlockSpec((1, _HIDDEN), lambda i: (0, 0)),
            pl.BlockSpec((_HIDDEN, _HIDDEN), lambda i: (0, 0)),
            pl.BlockSpec((1, _HIDDEN), lambda i: (0, 0)),
        ],
        out_specs=pl.BlockSpec((t, _HIDDEN), lambda i: (i, 0)),
        out_shape=jax.ShapeDtypeStruct((n, _HIDDEN), jnp.float32),
        compiler_params=pltpu.CompilerParams(
            dimension_semantics=("parallel",)),
    )(emb, num, w1e, w1n, b1, w2, b2)


def kernel(ids, numeric, table, W1, b1, W2, b2):
    B, L = ids.shape
    n = B * L
    emb = _sc_gather(table, ids.T.reshape(-1))
    num_sm = numeric.transpose(1, 0, 2).reshape(n, 4)
    bf = jnp.bfloat16
    out = _tc_mlp(emb, num_sm,
                  W1[:_EMBED].astype(bf), W1[_EMBED:].astype(bf),
                  b1.reshape(1, _HIDDEN), W2.astype(bf),
                  b2.reshape(1, _HIDDEN))
    return out.reshape(L, B, _HIDDEN).transpose(1, 0, 2)


# speedup vs baseline: 4.3363x; 1.1875x over previous
import functools

import jax
import jax.numpy as jnp
from jax import lax
from jax.experimental import pallas as pl
from jax.experimental.pallas import tpu as pltpu
from jax.experimental.pallas import tpu_sc as plsc

_EMBED = 128
_HIDDEN = 256
_GATHER_WINDOW = 128
_TOKEN_BLOCK = 8192
_NUMF = 4


def _sc_gather(table, flat_ids):
    n = flat_ids.shape[0]
    idx = flat_ids.reshape(1, n)
    mesh = plsc.VectorSubcoreMesh(core_axis_name="core",
                                  subcore_axis_name="subcore")

    @pl.kernel(out_type=jax.ShapeDtypeStruct((n, _EMBED), table.dtype),
               mesh=mesh)
    def gather_kernel(tab_hbm, i_hbm, o_hbm):
        def body(i_vmem, o_vmem):
            pltpu.sync_copy(tab_hbm.at[i_vmem.at[0]], o_vmem)

        pltpu.emit_pipeline(
            body,
            grid=(n // _GATHER_WINDOW,),
            in_specs=[pl.BlockSpec((1, _GATHER_WINDOW), lambda i: (0, i))],
            out_specs=[pl.BlockSpec((_GATHER_WINDOW, _EMBED),
                                    lambda i: (i, 0))],
            core_axis_name=("core", "subcore"),
            dimension_semantics=(pltpu.PARALLEL,),
        )(i_hbm, o_hbm)

    return gather_kernel(table, idx)


def _mlp_block_kernel(seqs_per_blk, emb_ref, numt_ref, w1e_ref, w1n_ref,
                      b1_ref, w2_ref, b2_ref, out_ref):
    h = jnp.dot(emb_ref[...].astype(jnp.bfloat16), w1e_ref[...],
                preferred_element_type=jnp.float32)
    nt = numt_ref[...].astype(jnp.bfloat16)
    w1n = w1n_ref[...]
    dn = (((0,), (0,)), ((), ()))
    hn = jnp.concatenate(
        [lax.dot_general(nt[_NUMF * j:_NUMF * (j + 1)], w1n, dn,
                         preferred_element_type=jnp.float32)
         for j in range(seqs_per_blk)],
        axis=0)
    h = jnp.maximum(h + hn + b1_ref[...], 0.0)
    o = jnp.dot(h.astype(jnp.bfloat16), w2_ref[...],
                preferred_element_type=jnp.float32)
    out_ref[...] = jnp.maximum(o + b2_ref[...], 0.0)


def _tc_mlp(emb, numt, w1e, w1n, b1, w2, b2):
    n = emb.shape[0]
    t = _TOKEN_BLOCK
    batch = numt.shape[1]
    seqs_per_blk = t // batch
    return pl.pallas_call(
        functools.partial(_mlp_block_kernel, seqs_per_blk),
        grid=(n // t,),
        in_specs=[
            pl.BlockSpec((t, _EMBED), lambda i: (i, 0)),
            pl.BlockSpec((_NUMF * seqs_per_blk, batch), lambda i: (i, 0)),
            pl.BlockSpec((_EMBED, _HIDDEN), lambda i: (0, 0)),
            pl.BlockSpec((_NUMF, _HIDDEN), lambda i: (0, 0)),
            pl.BlockSpec((1, _HIDDEN), lambda i: (0, 0)),
            pl.BlockSpec((_HIDDEN, _HIDDEN), lambda i: (0, 0)),
            pl.BlockSpec((1, _HIDDEN), lambda i: (0, 0)),
        ],
        out_specs=pl.BlockSpec((t, _HIDDEN), lambda i: (i, 0)),
        out_shape=jax.ShapeDtypeStruct((n, _HIDDEN), jnp.float32),
        compiler_params=pltpu.CompilerParams(
            dimension_semantics=("parallel",)),
    )(emb, numt, w1e, w1n, b1, w2, b2)


def kernel(ids, numeric, table, W1, b1, W2, b2):
    B, L = ids.shape
    n = B * L
    emb = _sc_gather(table, ids.T.reshape(-1))
    numt = numeric.reshape(B, L * _NUMF).T
    bf = jnp.bfloat16
    out = _tc_mlp(emb, numt,
                  W1[:_EMBED].astype(bf), W1[_EMBED:].astype(bf),
                  b1.reshape(1, _HIDDEN), W2.astype(bf),
                  b2.reshape(1, _HIDDEN))
    return out.reshape(L, B, _HIDDEN).transpose(1, 0, 2)
